# Initial kernel scaffold; baseline (speedup 1.0000x reference)
import jax
import jax.numpy as jnp
from jax.experimental import pallas as pl


def kernel(embeddings, labels, centers):
    raise NotImplementedError("write your pallas kernel here")


# speedup vs baseline: 1.3138x; 1.3138x over previous
import functools

import jax
import jax.numpy as jnp
from jax import lax
from jax.experimental import pallas as pl
from jax.experimental.pallas import tpu as pltpu
from jax.experimental.pallas import tpu_sc as plsc

NUM_CLASSES = 1000
FEAT_DIM = 512
LAMBDA_CENTER = 0.001
BATCH = 16384
NUM_WORKERS = 32
ROWS_PER_W = BATCH // NUM_WORKERS
CHUNK = 64
NCHUNK = ROWS_PER_W // CHUNK
LANES = 16
VECS_PER_ROW = FEAT_DIM // LANES


def _sc_partials(embeddings, labels3, centers):
    mesh = plsc.VectorSubcoreMesh(core_axis_name="c", subcore_axis_name="s")

    @functools.partial(
        pl.kernel,
        mesh=mesh,
        out_type=jax.ShapeDtypeStruct((NUM_WORKERS, LANES), jnp.float32),
        scratch_types=[
            pltpu.VMEM((NCHUNK, CHUNK), jnp.int32),
            pltpu.VMEM((CHUNK, FEAT_DIM), jnp.float32),
            pltpu.VMEM((CHUNK, FEAT_DIM), jnp.float32),
            pltpu.VMEM((LANES,), jnp.float32),
            pltpu.SemaphoreType.DMA,
            pltpu.SemaphoreType.DMA,
        ],
    )
    def k(e_hbm, l_hbm, c_hbm, out_hbm, idx_v, ebuf, cbuf, accv, sem1, sem2):
        wid = lax.axis_index("s") * 2 + lax.axis_index("c")
        base = wid * ROWS_PER_W
        pltpu.sync_copy(l_hbm.at[wid], idx_v)

        def chunk_body(ci, acc):
            cp1 = pltpu.async_copy(
                e_hbm.at[pl.ds(base + ci * CHUNK, CHUNK)], ebuf, sem1)
            cp2 = pltpu.async_copy(c_hbm.at[idx_v.at[ci]], cbuf, sem2)
            cp1.wait()
            cp2.wait()

            def row_body(r, acc_in):
                a = acc_in
                for j in range(VECS_PER_ROW):
                    e = ebuf[r, pl.ds(j * LANES, LANES)]
                    c = cbuf[r, pl.ds(j * LANES, LANES)]
                    d = e - c
                    a = a + d * d
                return a

            return lax.fori_loop(0, CHUNK, row_body, acc)

        acc = lax.fori_loop(0, NCHUNK, chunk_body,
                            jnp.zeros((LANES,), jnp.float32))
        accv[...] = acc
        pltpu.sync_copy(accv, out_hbm.at[wid])

    return k(embeddings, labels3, centers)


def _finalize(partials):
    def body(p_ref, o_ref):
        o_ref[0, 0] = jnp.sum(p_ref[...]) * (LAMBDA_CENTER / BATCH)

    out = pl.pallas_call(
        body,
        out_shape=jax.ShapeDtypeStruct((1, 1), jnp.float32),
        out_specs=pl.BlockSpec(memory_space=pltpu.SMEM),
    )(partials)
    return out[0, 0]


def kernel(embeddings, labels, centers):
    labels3 = labels.astype(jnp.int32).reshape(NUM_WORKERS, NCHUNK, CHUNK)
    partials = _sc_partials(embeddings, labels3, centers)
    return _finalize(partials)


# speedup vs baseline: 1.5241x; 1.1601x over previous
import functools

import jax
import jax.numpy as jnp
from jax import lax
from jax.experimental import pallas as pl
from jax.experimental.pallas import tpu as pltpu
from jax.experimental.pallas import tpu_sc as plsc

NUM_CLASSES = 1000
FEAT_DIM = 512
LAMBDA_CENTER = 0.001
BATCH = 16384
NUM_WORKERS = 32
ROWS_PER_W = BATCH // NUM_WORKERS
CHUNK = 32
NCHUNK = ROWS_PER_W // CHUNK
LANES = 16
VECS_PER_ROW = FEAT_DIM // LANES


def _sc_partials(embeddings, labels3, centers):
    mesh = plsc.VectorSubcoreMesh(core_axis_name="c", subcore_axis_name="s")

    @functools.partial(
        pl.kernel,
        mesh=mesh,
        out_type=jax.ShapeDtypeStruct((NUM_WORKERS, LANES), jnp.float32),
        scratch_types=[
            pltpu.VMEM((NCHUNK, CHUNK), jnp.int32),
            pltpu.VMEM((CHUNK, FEAT_DIM), jnp.float32),
            pltpu.VMEM((CHUNK, FEAT_DIM), jnp.float32),
            pltpu.VMEM((CHUNK, FEAT_DIM), jnp.float32),
            pltpu.VMEM((CHUNK, FEAT_DIM), jnp.float32),
            pltpu.VMEM((LANES,), jnp.float32),
            pltpu.SemaphoreType.DMA,
            pltpu.SemaphoreType.DMA,
            pltpu.SemaphoreType.DMA,
            pltpu.SemaphoreType.DMA,
        ],
    )
    def k(e_hbm, l_hbm, c_hbm, out_hbm, idx_v, eb0, eb1, cb0, cb1, accv,
          se0, se1, sc0, sc1):
        wid = lax.axis_index("s") * 2 + lax.axis_index("c")
        base = wid * ROWS_PER_W
        pltpu.sync_copy(l_hbm.at[wid], idx_v)
        ebufs = (eb0, eb1)
        cbufs = (cb0, cb1)
        sems_e = (se0, se1)
        sems_c = (sc0, sc1)

        def issue(ci):
            slot = ci % 2
            cpe = pltpu.async_copy(
                e_hbm.at[pl.ds(base + ci * CHUNK, CHUNK)],
                ebufs[slot], sems_e[slot])
            cpc = pltpu.async_copy(
                c_hbm.at[idx_v.at[ci]], cbufs[slot], sems_c[slot])
            return cpe, cpc

        pending = issue(0)
        acc = jnp.zeros((LANES,), jnp.float32)
        for ci in range(NCHUNK):
            nxt = issue(ci + 1) if ci + 1 < NCHUNK else None
            pending[0].wait()
            pending[1].wait()
            slot = ci % 2
            eb = ebufs[slot]
            cb = cbufs[slot]

            def row_body(r, a, eb=eb, cb=cb):
                for j in range(VECS_PER_ROW):
                    e = eb[r, pl.ds(j * LANES, LANES)]
                    c = cb[r, pl.ds(j * LANES, LANES)]
                    d = e - c
                    a = a + d * d
                return a

            acc = lax.fori_loop(0, CHUNK, row_body, acc)
            pending = nxt

        accv[...] = acc
        pltpu.sync_copy(accv, out_hbm.at[wid])

    return k(embeddings, labels3, centers)


def _finalize(partials):
    def body(p_ref, o_ref):
        o_ref[0, 0] = jnp.sum(p_ref[...]) * (LAMBDA_CENTER / BATCH)

    out = pl.pallas_call(
        body,
        out_shape=jax.ShapeDtypeStruct((1, 1), jnp.float32),
        out_specs=pl.BlockSpec(memory_space=pltpu.SMEM),
    )(partials)
    return out[0, 0]


def kernel(embeddings, labels, centers):
    labels3 = labels.astype(jnp.int32).reshape(NUM_WORKERS, NCHUNK, CHUNK)
    partials = _sc_partials(embeddings, labels3, centers)
    return _finalize(partials)


# speedup vs baseline: 1.5603x; 1.0237x over previous
import functools

import jax
import jax.numpy as jnp
from jax import lax
from jax.experimental import pallas as pl
from jax.experimental.pallas import tpu as pltpu
from jax.experimental.pallas import tpu_sc as plsc

NUM_CLASSES = 1000
FEAT_DIM = 512
LAMBDA_CENTER = 0.001
BATCH = 16384
NUM_WORKERS = 32
ROWS_PER_W = BATCH // NUM_WORKERS
CHUNK = 32
NCHUNK = ROWS_PER_W // CHUNK
LANES = 16
VECS_PER_ROW = FEAT_DIM // LANES


def _sc_partials(embeddings, labels3, centers):
    mesh = plsc.VectorSubcoreMesh(core_axis_name="c", subcore_axis_name="s")

    @functools.partial(
        pl.kernel,
        mesh=mesh,
        out_type=jax.ShapeDtypeStruct((NUM_WORKERS, LANES), jnp.float32),
        scratch_types=[
            pltpu.VMEM((NCHUNK, CHUNK), jnp.int32),
            pltpu.VMEM((CHUNK, FEAT_DIM), jnp.float32),
            pltpu.VMEM((CHUNK, FEAT_DIM), jnp.float32),
            pltpu.VMEM((CHUNK, FEAT_DIM), jnp.float32),
            pltpu.VMEM((CHUNK, FEAT_DIM), jnp.float32),
            pltpu.VMEM((LANES,), jnp.float32),
            pltpu.SemaphoreType.DMA,
            pltpu.SemaphoreType.DMA,
            pltpu.SemaphoreType.DMA,
            pltpu.SemaphoreType.DMA,
        ],
    )
    def k(e_hbm, l_hbm, c_hbm, out_hbm, idx_v, eb0, eb1, cb0, cb1, accv,
          se0, se1, sc0, sc1):
        wid = lax.axis_index("s") * 2 + lax.axis_index("c")
        base = wid * ROWS_PER_W
        pltpu.sync_copy(l_hbm.at[wid], idx_v)
        ebufs = (eb0, eb1)
        cbufs = (cb0, cb1)
        sems_e = (se0, se1)
        sems_c = (sc0, sc1)

        def issue(ci):
            slot = ci % 2
            cpe = pltpu.async_copy(
                e_hbm.at[pl.ds(base + ci * CHUNK, CHUNK)],
                ebufs[slot], sems_e[slot])
            cpc = pltpu.async_copy(
                c_hbm.at[idx_v.at[ci]], cbufs[slot], sems_c[slot])
            return cpe, cpc

        NACC = 8
        pending = issue(0)
        accs = tuple(jnp.zeros((LANES,), jnp.float32) for _ in range(NACC))
        for ci in range(NCHUNK):
            nxt = issue(ci + 1) if ci + 1 < NCHUNK else None
            pending[0].wait()
            pending[1].wait()
            slot = ci % 2
            eb = ebufs[slot]
            cb = cbufs[slot]

            def row_body(r, a, eb=eb, cb=cb):
                a = list(a)
                for j in range(VECS_PER_ROW):
                    e = eb[r, pl.ds(j * LANES, LANES)]
                    c = cb[r, pl.ds(j * LANES, LANES)]
                    d = e - c
                    a[j % NACC] = a[j % NACC] + d * d
                return tuple(a)

            accs = lax.fori_loop(0, CHUNK, row_body, accs)
            pending = nxt

        acc = accs[0]
        for i in range(1, NACC):
            acc = acc + accs[i]
        accv[...] = acc
        pltpu.sync_copy(accv, out_hbm.at[wid])

    return k(embeddings, labels3, centers)


def _finalize(partials):
    def body(p_ref, o_ref):
        o_ref[0, 0] = jnp.sum(p_ref[...]) * (LAMBDA_CENTER / BATCH)

    out = pl.pallas_call(
        body,
        out_shape=jax.ShapeDtypeStruct((1, 1), jnp.float32),
        out_specs=pl.BlockSpec(memory_space=pltpu.SMEM),
    )(partials)
    return out[0, 0]


def kernel(embeddings, labels, centers):
    labels3 = labels.astype(jnp.int32).reshape(NUM_WORKERS, NCHUNK, CHUNK)
    partials = _sc_partials(embeddings, labels3, centers)
    return _finalize(partials)


# speedup vs baseline: 1.8782x; 1.2038x over previous
import functools

import jax
import jax.numpy as jnp
from jax import lax
from jax.experimental import pallas as pl
from jax.experimental.pallas import tpu as pltpu
from jax.experimental.pallas import tpu_sc as plsc

NUM_CLASSES = 1000
FEAT_DIM = 512
LAMBDA_CENTER = 0.001
BATCH = 16384
NUM_WORKERS = 32
ROWS_PER_W = BATCH // NUM_WORKERS
CHUNK = 64
NCHUNK = ROWS_PER_W // CHUNK
LANES = 16
GROUPS_PER_ROW = FEAT_DIM // 32
NACC = 8


def _sc_partials(embeddings, labels3, centers_bf):
    mesh = plsc.VectorSubcoreMesh(core_axis_name="c", subcore_axis_name="s")

    @functools.partial(
        pl.kernel,
        mesh=mesh,
        out_type=jax.ShapeDtypeStruct((NUM_WORKERS, LANES), jnp.float32),
        compiler_params=pltpu.CompilerParams(needs_layout_passes=False),
        scratch_types=[
            pltpu.VMEM((NCHUNK, CHUNK), jnp.int32),
            pltpu.VMEM((CHUNK, FEAT_DIM), jnp.float32),
            pltpu.VMEM((CHUNK, FEAT_DIM), jnp.float32),
            pltpu.VMEM((CHUNK, FEAT_DIM // 2), jnp.int32),
            pltpu.VMEM((CHUNK, FEAT_DIM // 2), jnp.int32),
            pltpu.VMEM((LANES,), jnp.float32),
            pltpu.SemaphoreType.DMA,
            pltpu.SemaphoreType.DMA,
            pltpu.SemaphoreType.DMA,
            pltpu.SemaphoreType.DMA,
        ],
    )
    def k(e_hbm, l_hbm, c_hbm, out_hbm, idx_v, eb0, eb1, cb0, cb1, accv,
          se0, se1, sc0, sc1):
        wid = lax.axis_index("s") * 2 + lax.axis_index("c")
        base = wid * ROWS_PER_W
        pltpu.sync_copy(l_hbm.at[wid], idx_v)
        ebufs = (eb0, eb1)
        cbufs = (cb0, cb1)
        sems_e = (se0, se1)
        sems_c = (sc0, sc1)

        def issue(ci):
            slot = ci % 2
            cpe = pltpu.async_copy(
                e_hbm.at[pl.ds(base + ci * CHUNK, CHUNK)],
                ebufs[slot], sems_e[slot])
            cpc = pltpu.async_copy(
                c_hbm.at[idx_v.at[ci]], cbufs[slot], sems_c[slot])
            return cpe, cpc

        pending = issue(0)
        accs = tuple(jnp.zeros((LANES,), jnp.float32) for _ in range(NACC))
        for ci in range(NCHUNK):
            nxt = issue(ci + 1) if ci + 1 < NCHUNK else None
            pending[0].wait()
            pending[1].wait()
            slot = ci % 2
            eb = ebufs[slot]
            cb = cbufs[slot]

            def row_body(r, a, eb=eb, cb=cb):
                a = list(a)
                for g in range(GROUPS_PER_ROW):
                    c32i = cb[r, pl.ds(g * LANES, LANES)]
                    c32 = plsc.bitcast(c32i, jnp.bfloat16)
                    c_lo, c_hi = plsc.unpack(
                        c32, format=plsc.PackFormat.INTERLEAVED,
                        preferred_element_type=jnp.float32)
                    e_lo = eb[r, pl.ds(g * 32, LANES)]
                    e_hi = eb[r, pl.ds(g * 32 + LANES, LANES)]
                    d1 = e_lo - c_lo
                    d2 = e_hi - c_hi
                    a[(2 * g) % NACC] = a[(2 * g) % NACC] + d1 * d1
                    a[(2 * g + 1) % NACC] = a[(2 * g + 1) % NACC] + d2 * d2
                return tuple(a)

            accs = lax.fori_loop(0, CHUNK, row_body, accs)
            pending = nxt

        acc = accs[0]
        for i in range(1, NACC):
            acc = acc + accs[i]
        accv[...] = acc
        pltpu.sync_copy(accv, out_hbm.at[wid])

    return k(embeddings, labels3, centers_bf)


def _finalize(partials):
    def body(p_ref, o_ref):
        o_ref[0, 0] = jnp.sum(p_ref[...]) * (LAMBDA_CENTER / BATCH)

    out = pl.pallas_call(
        body,
        out_shape=jax.ShapeDtypeStruct((1, 1), jnp.float32),
        out_specs=pl.BlockSpec(memory_space=pltpu.SMEM),
    )(partials)
    return out[0, 0]


def kernel(embeddings, labels, centers):
    labels3 = labels.astype(jnp.int32).reshape(NUM_WORKERS, NCHUNK, CHUNK)
    centers_bf = (centers.astype(jnp.bfloat16)
                  .reshape(NUM_CLASSES, 16, 2, LANES)
                  .swapaxes(2, 3)
                  .reshape(NUM_CLASSES, FEAT_DIM // 2, 2))
    centers_bf = lax.bitcast_convert_type(centers_bf, jnp.int32)
    partials = _sc_partials(embeddings, labels3, centers_bf)
    return _finalize(partials)
